# Initial kernel scaffold; baseline (speedup 1.0000x reference)
import jax
import jax.numpy as jnp
from jax.experimental import pallas as pl


def kernel(x):
    raise NotImplementedError("write your pallas kernel here")


# speedup vs baseline: 29.2231x; 29.2231x over previous
import functools

import jax
import jax.numpy as jnp
from jax.experimental import pallas as pl
from jax.experimental.pallas import tpu as pltpu

_WL = 8
_LOSS_THR = 1.0
_C = 32
_H = 512
_W = 512
_NB = _W // _WL


def _minmax_body(x_ref, mx_ref, mn_ref):
    i = pl.program_id(0)
    t = x_ref[...]
    xl = jnp.concatenate(
        [jnp.zeros((t.shape[0], t.shape[1], 1), t.dtype), t[:, :, :-1]], axis=2)
    xd = t - xl
    m = jnp.max(xd)
    n = jnp.min(xd)

    @pl.when(i == 0)
    def _():
        mx_ref[0, 0] = m
        mn_ref[0, 0] = n

    @pl.when(i > 0)
    def _():
        mx_ref[0, 0] = jnp.maximum(mx_ref[0, 0], m)
        mn_ref[0, 0] = jnp.minimum(mn_ref[0, 0], n)


def _seg_reduce8(a, op, pad):
    b = op(a, jnp.roll(a, -1, axis=-1))
    b = op(b, jnp.roll(b, -2, axis=-1))
    b = op(b, jnp.roll(b, -4, axis=-1))
    lane = jax.lax.broadcasted_iota(jnp.int32, a.shape, a.ndim - 1)
    e = jnp.where(lane % 8 == 0, b, pad)
    f = op(e, jnp.roll(e, 1, axis=-1))
    f = op(f, jnp.roll(f, 2, axis=-1))
    f = op(f, jnp.roll(f, 4, axis=-1))
    return f


def _fit_body(s_ref, x_ref, o_ref):
    mn = s_ref[0, 0]
    sc = s_ref[0, 1]
    isc = s_ref[0, 2]
    lsb = s_ref[0, 3]
    ilsb = s_ref[0, 4]

    t = x_ref[...]
    xl = jnp.concatenate(
        [jnp.zeros((_C, _WL, 1), t.dtype), t[:, :, :-1]], axis=2)
    xd = t - xl
    v = jnp.round((xd - mn) * sc) * isc + mn

    a1 = v[0]
    a2 = v[1]

    wi = jax.lax.broadcasted_iota(jnp.int32, (_W, _NB), 0)
    bi = jax.lax.broadcasted_iota(jnp.int32, (_W, _NB), 1)
    S = (wi // _WL == bi).astype(jnp.float32)
    bi2 = jax.lax.broadcasted_iota(jnp.int32, (_NB, _W), 0)
    wi2 = jax.lax.broadcasted_iota(jnp.int32, (_NB, _W), 1)
    E = (wi2 // _WL == bi2).astype(jnp.float32)

    def bsum(z):
        return jnp.dot(z.sum(axis=1), S, preferred_element_type=jnp.float32)

    B3 = bsum(v)
    B1 = bsum(v * a1[None])
    B2 = bsum(v * a2[None])
    s1 = B3[0]
    s2 = B3[1]
    s11 = B1[0]
    s12 = B1[1]
    s22 = B2[1]
    n = jnp.float32(_WL * _WL)

    det = (s11 * (s22 * n - s2 * s2)
           - s12 * (s12 * n - s2 * s1)
           + s1 * (s12 * s2 - s22 * s1))
    sing = det == 0.0
    idet = 1.0 / jnp.where(sing, 1.0, det)
    i00 = jnp.where(sing, 1.0, (s22 * n - s2 * s2) * idet)
    i01 = jnp.where(sing, 0.0, (s1 * s2 - s12 * n) * idet)
    i02 = jnp.where(sing, 0.0, (s12 * s2 - s22 * s1) * idet)
    i11 = jnp.where(sing, 1.0, (s11 * n - s1 * s1) * idet)
    i12 = jnp.where(sing, 0.0, (s12 * s1 - s11 * s2) * idet)
    i22 = jnp.where(sing, 1.0, (s11 * s22 - s12 * s12) * idet)

    def san(c):
        return jnp.nan_to_num(c, nan=1e15, posinf=1e15, neginf=-1e15)

    c0 = san(i00[None] * B1 + i01[None] * B2 + i02[None] * B3)
    c1 = san(i01[None] * B1 + i11[None] * B2 + i12[None] * B3)
    c2 = san(i02[None] * B1 + i12[None] * B2 + i22[None] * B3)

    C0 = jnp.dot(c0, E, preferred_element_type=jnp.float32)
    C1 = jnp.dot(c1, E, preferred_element_type=jnp.float32)
    C2 = jnp.dot(c2, E, preferred_element_type=jnp.float32)

    r = C0[:, None, :] * a1[None] + C1[:, None, :] * a2[None] + C2[:, None, :]
    r1 = jnp.round(r * ilsb) * lsb

    L = jnp.dot(((v - r1) ** 2).sum(axis=1), S,
                preferred_element_type=jnp.float32)
    LE = jnp.dot(L, E, preferred_element_type=jnp.float32)

    mx1 = _seg_reduce8(a1, jnp.maximum, jnp.float32(-jnp.inf)).max(axis=0)
    mn1 = _seg_reduce8(a1, jnp.minimum, jnp.float32(jnp.inf)).min(axis=0)
    mx2 = _seg_reduce8(a2, jnp.maximum, jnp.float32(-jnp.inf)).max(axis=0)
    mn2 = _seg_reduce8(a2, jnp.minimum, jnp.float32(jnp.inf)).min(axis=0)
    cmask = (mx1 - mn1 < 1e-6) & (mx2 - mn2 < 1e-6)

    LE = jnp.where(cmask[None, :], _LOSS_THR + 1.0, LE)
    take_fit = LE <= _LOSS_THR
    rr = jnp.where(take_fit[:, None, :], r1, v)
    o_ref[...] = rr + xl


@functools.partial(jax.jit, static_argnames=())
def kernel(x):
    x2 = x[0]

    mx, mn = pl.pallas_call(
        _minmax_body,
        grid=(8,),
        in_specs=[pl.BlockSpec((_C, _H // 8, _W), lambda i: (0, i, 0))],
        out_specs=[
            pl.BlockSpec((1, 1), lambda i: (0, 0), memory_space=pltpu.SMEM),
            pl.BlockSpec((1, 1), lambda i: (0, 0), memory_space=pltpu.SMEM),
        ],
        out_shape=[
            jax.ShapeDtypeStruct((1, 1), jnp.float32),
            jax.ShapeDtypeStruct((1, 1), jnp.float32),
        ],
    )(x2)
    mx = mx[0, 0]
    mn = mn[0, 0]

    scale = (2.0 ** 16 - 1.0) / (mx - mn)
    iscale = 1.0 / scale
    lsb = 2.0 ** (jnp.round(jnp.log2(mx / 2.0 ** 15)) + 1.0)
    ilsb = 1.0 / lsb
    scalars = jnp.stack([mn, scale, iscale, lsb, ilsb,
                         jnp.float32(0), jnp.float32(0), jnp.float32(0)])
    scalars = scalars.astype(jnp.float32).reshape(1, 8)

    out = pl.pallas_call(
        _fit_body,
        grid=(_H // _WL,),
        in_specs=[
            pl.BlockSpec(memory_space=pltpu.SMEM),
            pl.BlockSpec((_C, _WL, _W), lambda i: (0, i, 0)),
        ],
        out_specs=pl.BlockSpec((_C, _WL, _W), lambda i: (0, i, 0)),
        out_shape=jax.ShapeDtypeStruct((_C, _H, _W), jnp.float32),
    )(scalars, x2)
    return out[None]
